# baseline (speedup 1.0000x reference)
import dataclasses
import functools

import jax
import jax.numpy as jnp
from jax import lax
from jax.experimental import pallas as pl
from jax.experimental.pallas import tpu as pltpu
from jax.experimental.pallas import tpu_sc as plsc

B = 2
C = 128
N = 10000
K = 16
G = 4
CG = C // G
BN = B * N
DA = 144
LANES = 16

NW = 32
T = 8
NBUF = 2
BNP = ((BN + NW * T * NBUF - 1) // (NW * T * NBUF)) * NW * T * NBUF
ROWS_PER_W = BNP // NW
CHUNKS = ROWS_PER_W // T

NBLK = 1000
NB = N // NBLK


def _tc_feats_body(f_ref, w1_ref, w2_ref, edge_ref, loc_ref):
    f = f_ref[0]
    dn = (((0,), (1,)), ((), ()))
    edge_ref[...] = lax.dot_general(f, w2_ref[...], dn,
                                    precision=lax.Precision.HIGHEST,
                                    preferred_element_type=jnp.float32)
    loc_ref[...] = lax.dot_general(f, w1_ref[...], dn,
                                   precision=lax.Precision.HIGHEST,
                                   preferred_element_type=jnp.float32)


def _tc_feats(feature, w1, w2):
    return pl.pallas_call(
        _tc_feats_body,
        grid=(B,),
        in_specs=[
            pl.BlockSpec((1, C, N), lambda b: (b, 0, 0)),
            pl.BlockSpec((C, C), lambda b: (0, 0)),
            pl.BlockSpec((C, C), lambda b: (0, 0)),
        ],
        out_specs=[
            pl.BlockSpec((N, C), lambda b: (b, 0)),
            pl.BlockSpec((N, C), lambda b: (b, 0)),
        ],
        out_shape=[
            jax.ShapeDtypeStruct((BN, C), jnp.float32),
            jax.ShapeDtypeStruct((BN, C), jnp.float32),
        ],
    )(feature, w1, w2)


def _sc_compiler_params():
    cp = pltpu.CompilerParams()
    if "needs_layout_passes" in pltpu.CompilerParams.__dataclass_fields__:
        cp = dataclasses.replace(cp, needs_layout_passes=False)
    return cp


@functools.cache
def _sc_gather_sum_kernel():
    mesh = plsc.VectorSubcoreMesh(core_axis_name="c", subcore_axis_name="s")

    @functools.partial(
        pl.kernel,
        out_type=jax.ShapeDtypeStruct((BNP, DA), jnp.float32),
        mesh=mesh,
        compiler_params=_sc_compiler_params(),
        scratch_types=[
            pltpu.VMEM((T * K,), jnp.int32),
            pltpu.VMEM((T * K, C), jnp.float32),
            pltpu.VMEM((T, DA), jnp.float32),
        ],
    )
    def sc_gather_sum(edge_hbm, idx_hbm, out_hbm, idx_v, rows_v, acc_v):
        wid = lax.axis_index("s") * 2 + lax.axis_index("c")
        base0 = wid * ROWS_PER_W

        @pl.loop(0, CHUNKS)
        def _chunk(i):
            base = base0 + i * T
            pltpu.sync_copy(idx_hbm.at[pl.ds(base * K, T * K)], idx_v)
            pltpu.sync_copy(edge_hbm.at[idx_v], rows_v)
            for t in range(T):
                qsums = []
                for j in range(C // LANES):
                    sl = pl.ds(j * LANES, LANES)
                    vals = [rows_v[t * K + k, sl] for k in range(K)]
                    sqs = [v * v for v in vals]
                    while len(vals) > 1:
                        vals = [vals[p] + vals[p + 1]
                                for p in range(0, len(vals), 2)]
                        sqs = [sqs[p] + sqs[p + 1]
                               for p in range(0, len(sqs), 2)]
                    acc_v[t, sl] = vals[0]
                    qsums.append(jnp.sum(sqs[0]))
                lane = lax.iota(jnp.int32, LANES)
                qv = jnp.zeros((LANES,), jnp.float32)
                for g in range(G):
                    qg = qsums[2 * g] + qsums[2 * g + 1]
                    qv = qv + jnp.where(lane == g, qg, 0.0)
                acc_v[t, pl.ds(C, LANES)] = qv
            pltpu.sync_copy(acc_v, out_hbm.at[pl.ds(base, T)])

    return sc_gather_sum


def _sc_gather_sum(edge, idx_flat):
    return _sc_gather_sum_kernel()(edge, idx_flat)


def _tc_stats_body(s_ref, l_ref, gam_ref, bet_ref, g_ref, gt_ref,
                   a_ref, c0_ref, acc_s, acc_q, acc_l, acc_x, acc_2):
    j = pl.program_id(1)
    s = s_ref[:, 0:C]
    q = s_ref[:, C:C + G]
    l = l_ref[...]
    ps = jnp.sum(s, axis=0, keepdims=True)
    pq = jnp.sum(q, axis=0, keepdims=True)
    pl_ = jnp.sum(l, axis=0, keepdims=True)
    px = jnp.sum(l * s, axis=0, keepdims=True)
    p2 = jnp.sum(l * l, axis=0, keepdims=True)

    @pl.when(j == 0)
    def _():
        acc_s[...] = ps
        acc_q[...] = pq
        acc_l[...] = pl_
        acc_x[...] = px
        acc_2[...] = p2

    @pl.when(j > 0)
    def _():
        acc_s[...] += ps
        acc_q[...] += pq
        acc_l[...] += pl_
        acc_x[...] += px
        acc_2[...] += p2

    @pl.when(j == NB - 1)
    def _():
        gm = g_ref[...]
        dn = (((1,), (0,)), ((), ()))
        hi = lax.Precision.HIGHEST
        t1 = lax.dot_general(acc_s[...], gm, dn, precision=hi,
                             preferred_element_type=jnp.float32)
        lg = lax.dot_general(acc_l[...], gm, dn, precision=hi,
                             preferred_element_type=jnp.float32)
        xg = lax.dot_general(acc_x[...], gm, dn, precision=hi,
                             preferred_element_type=jnp.float32)
        l2 = lax.dot_general(acc_2[...], gm, dn, precision=hi,
                             preferred_element_type=jnp.float32)
        qg = acc_q[...]
        cnt = float(CG * N * K)
        mean4 = (t1 - K * lg) / cnt
        e2 = (qg - 2.0 * xg + K * l2) / cnt
        var4 = e2 - mean4 * mean4
        inv4 = lax.rsqrt(var4 + 1e-5)
        gt = gt_ref[...]
        mean_c = lax.dot_general(mean4, gt, dn, precision=hi,
                                 preferred_element_type=jnp.float32)
        inv_c = lax.dot_general(inv4, gt, dn, precision=hi,
                                preferred_element_type=jnp.float32)
        a = gam_ref[...] * inv_c
        a_ref[0] = jnp.broadcast_to(a, (8, C))
        c0_ref[0] = jnp.broadcast_to(bet_ref[...] - a * mean_c, (8, C))


def _tc_stats(s_aug, loc, gam, bet, gmat, gmat_t):
    return pl.pallas_call(
        _tc_stats_body,
        grid=(B, NB),
        in_specs=[
            pl.BlockSpec((NBLK, DA), lambda b, j: (b * NB + j, 0)),
            pl.BlockSpec((NBLK, C), lambda b, j: (b * NB + j, 0)),
            pl.BlockSpec((1, C), lambda b, j: (0, 0)),
            pl.BlockSpec((1, C), lambda b, j: (0, 0)),
            pl.BlockSpec((C, G), lambda b, j: (0, 0)),
            pl.BlockSpec((G, C), lambda b, j: (0, 0)),
        ],
        out_specs=[
            pl.BlockSpec((1, 8, C), lambda b, j: (b, 0, 0)),
            pl.BlockSpec((1, 8, C), lambda b, j: (b, 0, 0)),
        ],
        out_shape=[
            jax.ShapeDtypeStruct((B, 8, C), jnp.float32),
            jax.ShapeDtypeStruct((B, 8, C), jnp.float32),
        ],
        scratch_shapes=[
            pltpu.VMEM((1, C), jnp.float32),
            pltpu.VMEM((1, G), jnp.float32),
            pltpu.VMEM((1, C), jnp.float32),
            pltpu.VMEM((1, C), jnp.float32),
            pltpu.VMEM((1, C), jnp.float32),
        ],
    )(s_aug, loc, gam, bet, gmat, gmat_t)


def _tc_tables_body(e_ref, l_ref, a_ref, c0_ref, es_ref, t_ref):
    a = a_ref[0, 0:1, :]
    c0 = c0_ref[0, 0:1, :]
    es_ref[...] = e_ref[...] * (a * (1.0 / K))
    t_ref[...] = (c0 - a * l_ref[...]) * (1.0 / K)


def _tc_tables(edge, loc, a, c0):
    return pl.pallas_call(
        _tc_tables_body,
        grid=(B, NB),
        in_specs=[
            pl.BlockSpec((NBLK, C), lambda b, j: (b * NB + j, 0)),
            pl.BlockSpec((NBLK, C), lambda b, j: (b * NB + j, 0)),
            pl.BlockSpec((1, 8, C), lambda b, j: (b, 0, 0)),
            pl.BlockSpec((1, 8, C), lambda b, j: (b, 0, 0)),
        ],
        out_specs=[
            pl.BlockSpec((NBLK, C), lambda b, j: (b * NB + j, 0)),
            pl.BlockSpec((NBLK, C), lambda b, j: (b * NB + j, 0)),
        ],
        out_shape=[
            jax.ShapeDtypeStruct((BN, C), jnp.float32),
            jax.ShapeDtypeStruct((BNP, C), jnp.float32),
        ],
    )(edge, loc, a, c0)


@functools.cache
def _sc_edge_out_kernel():
    mesh = plsc.VectorSubcoreMesh(core_axis_name="c", subcore_axis_name="s")

    @functools.partial(
        pl.kernel,
        out_type=jax.ShapeDtypeStruct((BNP, C), jnp.float32),
        mesh=mesh,
        compiler_params=_sc_compiler_params(),
        scratch_types=[
            pltpu.VMEM((T * K,), jnp.int32),
            pltpu.VMEM((T * K, C), jnp.float32),
            pltpu.VMEM((T, C), jnp.float32),
            pltpu.VMEM((T, C), jnp.float32),
        ],
    )
    def sc_edge_out(es_hbm, t_hbm, idx_hbm, out_hbm, idx_v, rows_v, t_v, o_v):
        wid = lax.axis_index("s") * 2 + lax.axis_index("c")
        base0 = wid * ROWS_PER_W

        @pl.loop(0, CHUNKS)
        def _chunk(i):
            base = base0 + i * T
            pltpu.sync_copy(idx_hbm.at[pl.ds(base * K, T * K)], idx_v)
            pltpu.sync_copy(t_hbm.at[pl.ds(base, T)], t_v)
            pltpu.sync_copy(es_hbm.at[idx_v], rows_v)
            zero = jnp.zeros((LANES,), jnp.float32)
            for t in range(T):
                for j in range(C // LANES):
                    sl = pl.ds(j * LANES, LANES)
                    tj = t_v[t, sl]
                    vals = [jnp.maximum(rows_v[t * K + k, sl] + tj, zero)
                            for k in range(K)]
                    while len(vals) > 1:
                        vals = [vals[p] + vals[p + 1]
                                for p in range(0, len(vals), 2)]
                    o_v[t, sl] = vals[0]
            pltpu.sync_copy(o_v, out_hbm.at[pl.ds(base, T)])

    return sc_edge_out


def _sc_edge_out(es, tt, idx_flat):
    return _sc_edge_out_kernel()(es, tt, idx_flat)


def kernel(feature, knn_inds, W1, W2, gamma, beta):
    f32 = jnp.float32
    feature = feature.astype(f32)
    w1 = W1.astype(f32)
    w2 = W2.astype(f32)
    gam = gamma.astype(f32).reshape(1, C)
    bet = beta.astype(f32).reshape(1, C)
    gmat = (jnp.arange(C)[:, None] // CG
            == jnp.arange(G)[None, :]).astype(f32)
    gmat_t = gmat.T
    offs = (jnp.arange(B, dtype=jnp.int32) * N)[:, None, None]
    idx_flat = (knn_inds.astype(jnp.int32) + offs).reshape(BN * K)
    idx_pad = jnp.concatenate(
        [idx_flat, jnp.zeros(((BNP - BN) * K,), jnp.int32)])

    edge, loc = _tc_feats(feature, w1, w2)
    s_aug = _sc_gather_sum(edge, idx_pad)
    a, c0 = _tc_stats(s_aug, loc, gam, bet, gmat, gmat_t)
    es, tt = _tc_tables(edge, loc, a, c0)
    out_t = _sc_edge_out(es, tt, idx_pad)
    return out_t[:BN].reshape(B, N, C).transpose(0, 2, 1)


# speedup vs baseline: 2.1160x; 2.1160x over previous
import dataclasses
import functools

import jax
import jax.numpy as jnp
from jax import lax
from jax.experimental import pallas as pl
from jax.experimental.pallas import tpu as pltpu
from jax.experimental.pallas import tpu_sc as plsc

B = 2
C = 128
N = 10000
K = 16
G = 4
CG = C // G
BN = B * N
DA = 144
LANES = 16

NW = 32
T = 8
NBUF = 2
BNP = ((BN + NW * T * NBUF - 1) // (NW * T * NBUF)) * NW * T * NBUF
ROWS_PER_W = BNP // NW
CHUNKS = ROWS_PER_W // T

NBLK = 1000
NB = N // NBLK


def _tc_feats_body(f_ref, w1_ref, w2_ref, edge_ref, loc_ref):
    f = f_ref[0]
    dn = (((0,), (1,)), ((), ()))
    edge_ref[...] = lax.dot_general(f, w2_ref[...], dn,
                                    precision=lax.Precision.HIGHEST,
                                    preferred_element_type=jnp.float32)
    loc_ref[...] = lax.dot_general(f, w1_ref[...], dn,
                                   precision=lax.Precision.HIGHEST,
                                   preferred_element_type=jnp.float32)


def _tc_feats(feature, w1, w2):
    return pl.pallas_call(
        _tc_feats_body,
        grid=(B,),
        in_specs=[
            pl.BlockSpec((1, C, N), lambda b: (b, 0, 0)),
            pl.BlockSpec((C, C), lambda b: (0, 0)),
            pl.BlockSpec((C, C), lambda b: (0, 0)),
        ],
        out_specs=[
            pl.BlockSpec((N, C), lambda b: (b, 0)),
            pl.BlockSpec((N, C), lambda b: (b, 0)),
        ],
        out_shape=[
            jax.ShapeDtypeStruct((BN, C), jnp.float32),
            jax.ShapeDtypeStruct((BN, C), jnp.float32),
        ],
    )(feature, w1, w2)


def _sc_compiler_params():
    cp = pltpu.CompilerParams()
    if "needs_layout_passes" in pltpu.CompilerParams.__dataclass_fields__:
        cp = dataclasses.replace(cp, needs_layout_passes=False)
    return cp


@functools.cache
def _sc_gather_sum_kernel():
    mesh = plsc.VectorSubcoreMesh(core_axis_name="c", subcore_axis_name="s")

    @functools.partial(
        pl.kernel,
        out_type=jax.ShapeDtypeStruct((BNP, DA), jnp.float32),
        mesh=mesh,
        compiler_params=_sc_compiler_params(),
        scratch_types=[
            pltpu.VMEM((T * K,), jnp.int32),
            pltpu.VMEM((T * K, C), jnp.float32),
            pltpu.VMEM((T, DA), jnp.float32),
        ],
    )
    def sc_gather_sum(edge_hbm, idx_hbm, out_hbm, idx_v, rows_v, acc_v):
        wid = lax.axis_index("s") * 2 + lax.axis_index("c")
        base0 = wid * ROWS_PER_W

        @pl.loop(0, CHUNKS)
        def _chunk(i):
            base = base0 + i * T
            pltpu.sync_copy(idx_hbm.at[pl.ds(base * K, T * K)], idx_v)
            pltpu.sync_copy(edge_hbm.at[idx_v], rows_v)
            for t in range(T):
                qsums = []
                for j in range(C // LANES):
                    sl = pl.ds(j * LANES, LANES)
                    vals = [rows_v[t * K + k, sl] for k in range(K)]
                    sqs = [v * v for v in vals]
                    while len(vals) > 1:
                        vals = [vals[p] + vals[p + 1]
                                for p in range(0, len(vals), 2)]
                        sqs = [sqs[p] + sqs[p + 1]
                               for p in range(0, len(sqs), 2)]
                    acc_v[t, sl] = vals[0]
                    qsums.append(jnp.sum(sqs[0]))
                lane = lax.iota(jnp.int32, LANES)
                qv = jnp.zeros((LANES,), jnp.float32)
                for g in range(G):
                    qg = qsums[2 * g] + qsums[2 * g + 1]
                    qv = qv + jnp.where(lane == g, qg, 0.0)
                acc_v[t, pl.ds(C, LANES)] = qv
            pltpu.sync_copy(acc_v, out_hbm.at[pl.ds(base, T)])

    return sc_gather_sum


def _sc_gather_sum(edge, idx_flat):
    return _sc_gather_sum_kernel()(edge, idx_flat)


def _tc_stats_body(s_ref, l_ref, gam_ref, bet_ref, g_ref, gt_ref,
                   a_ref, c0_ref, acc_s, acc_q, acc_l, acc_x, acc_2):
    j = pl.program_id(1)
    s = s_ref[:, 0:C]
    q = s_ref[:, C:C + G]
    l = l_ref[...]
    ps = jnp.sum(s, axis=0, keepdims=True)
    pq = jnp.sum(q, axis=0, keepdims=True)
    pl_ = jnp.sum(l, axis=0, keepdims=True)
    px = jnp.sum(l * s, axis=0, keepdims=True)
    p2 = jnp.sum(l * l, axis=0, keepdims=True)

    @pl.when(j == 0)
    def _():
        acc_s[...] = ps
        acc_q[...] = pq
        acc_l[...] = pl_
        acc_x[...] = px
        acc_2[...] = p2

    @pl.when(j > 0)
    def _():
        acc_s[...] += ps
        acc_q[...] += pq
        acc_l[...] += pl_
        acc_x[...] += px
        acc_2[...] += p2

    @pl.when(j == NB - 1)
    def _():
        gm = g_ref[...]
        dn = (((1,), (0,)), ((), ()))
        hi = lax.Precision.HIGHEST
        t1 = lax.dot_general(acc_s[...], gm, dn, precision=hi,
                             preferred_element_type=jnp.float32)
        lg = lax.dot_general(acc_l[...], gm, dn, precision=hi,
                             preferred_element_type=jnp.float32)
        xg = lax.dot_general(acc_x[...], gm, dn, precision=hi,
                             preferred_element_type=jnp.float32)
        l2 = lax.dot_general(acc_2[...], gm, dn, precision=hi,
                             preferred_element_type=jnp.float32)
        qg = acc_q[...]
        cnt = float(CG * N * K)
        mean4 = (t1 - K * lg) / cnt
        e2 = (qg - 2.0 * xg + K * l2) / cnt
        var4 = e2 - mean4 * mean4
        inv4 = lax.rsqrt(var4 + 1e-5)
        gt = gt_ref[...]
        mean_c = lax.dot_general(mean4, gt, dn, precision=hi,
                                 preferred_element_type=jnp.float32)
        inv_c = lax.dot_general(inv4, gt, dn, precision=hi,
                                preferred_element_type=jnp.float32)
        a = gam_ref[...] * inv_c
        a_ref[0] = jnp.broadcast_to(a, (8, C))
        c0_ref[0] = jnp.broadcast_to(bet_ref[...] - a * mean_c, (8, C))


def _tc_stats(s_aug, loc, gam, bet, gmat, gmat_t):
    return pl.pallas_call(
        _tc_stats_body,
        grid=(B, NB),
        in_specs=[
            pl.BlockSpec((NBLK, DA), lambda b, j: (b * NB + j, 0)),
            pl.BlockSpec((NBLK, C), lambda b, j: (b * NB + j, 0)),
            pl.BlockSpec((1, C), lambda b, j: (0, 0)),
            pl.BlockSpec((1, C), lambda b, j: (0, 0)),
            pl.BlockSpec((C, G), lambda b, j: (0, 0)),
            pl.BlockSpec((G, C), lambda b, j: (0, 0)),
        ],
        out_specs=[
            pl.BlockSpec((1, 8, C), lambda b, j: (b, 0, 0)),
            pl.BlockSpec((1, 8, C), lambda b, j: (b, 0, 0)),
        ],
        out_shape=[
            jax.ShapeDtypeStruct((B, 8, C), jnp.float32),
            jax.ShapeDtypeStruct((B, 8, C), jnp.float32),
        ],
        scratch_shapes=[
            pltpu.VMEM((1, C), jnp.float32),
            pltpu.VMEM((1, G), jnp.float32),
            pltpu.VMEM((1, C), jnp.float32),
            pltpu.VMEM((1, C), jnp.float32),
            pltpu.VMEM((1, C), jnp.float32),
        ],
    )(s_aug, loc, gam, bet, gmat, gmat_t)


def _tc_tables_body(e_ref, l_ref, a_ref, c0_ref, es_ref, t_ref):
    a = a_ref[0, 0:1, :]
    c0 = c0_ref[0, 0:1, :]
    es_ref[...] = e_ref[...] * (a * (1.0 / K))
    t_ref[...] = (c0 - a * l_ref[...]) * (1.0 / K)


def _tc_tables(edge, loc, a, c0):
    return pl.pallas_call(
        _tc_tables_body,
        grid=(B, NB),
        in_specs=[
            pl.BlockSpec((NBLK, C), lambda b, j: (b * NB + j, 0)),
            pl.BlockSpec((NBLK, C), lambda b, j: (b * NB + j, 0)),
            pl.BlockSpec((1, 8, C), lambda b, j: (b, 0, 0)),
            pl.BlockSpec((1, 8, C), lambda b, j: (b, 0, 0)),
        ],
        out_specs=[
            pl.BlockSpec((NBLK, C), lambda b, j: (b * NB + j, 0)),
            pl.BlockSpec((NBLK, C), lambda b, j: (b * NB + j, 0)),
        ],
        out_shape=[
            jax.ShapeDtypeStruct((BN, C), jnp.float32),
            jax.ShapeDtypeStruct((BNP, C), jnp.float32),
        ],
    )(edge, loc, a, c0)


@functools.cache
def _sc_edge_out_kernel():
    mesh = plsc.VectorSubcoreMesh(core_axis_name="c", subcore_axis_name="s")

    @functools.partial(
        pl.kernel,
        out_type=jax.ShapeDtypeStruct((BNP, C), jnp.float32),
        mesh=mesh,
        compiler_params=_sc_compiler_params(),
        scratch_types=[
            pltpu.VMEM((T * K,), jnp.int32),
            pltpu.VMEM((T * K, C), jnp.float32),
            pltpu.VMEM((T, C), jnp.float32),
            pltpu.VMEM((T, C), jnp.float32),
        ],
    )
    def sc_edge_out(es_hbm, t_hbm, idx_hbm, out_hbm, idx_v, rows_v, t_v, o_v):
        wid = lax.axis_index("s") * 2 + lax.axis_index("c")
        base0 = wid * ROWS_PER_W

        @pl.loop(0, CHUNKS)
        def _chunk(i):
            base = base0 + i * T
            pltpu.sync_copy(idx_hbm.at[pl.ds(base * K, T * K)], idx_v)
            pltpu.sync_copy(t_hbm.at[pl.ds(base, T)], t_v)
            pltpu.sync_copy(es_hbm.at[idx_v], rows_v)
            zero = jnp.zeros((LANES,), jnp.float32)
            for t in range(T):
                for j in range(C // LANES):
                    sl = pl.ds(j * LANES, LANES)
                    tj = t_v[t, sl]
                    vals = [jnp.maximum(rows_v[t * K + k, sl] + tj, zero)
                            for k in range(K)]
                    while len(vals) > 1:
                        vals = [vals[p] + vals[p + 1]
                                for p in range(0, len(vals), 2)]
                    o_v[t, sl] = vals[0]
            pltpu.sync_copy(o_v, out_hbm.at[pl.ds(base, T)])

    return sc_edge_out


def _sc_edge_out(es, tt, idx_flat):
    return _sc_edge_out_kernel()(es, tt, idx_flat)


def kernel(feature, knn_inds, W1, W2, gamma, beta):
    f32 = jnp.float32
    feature = feature.astype(f32)
    w1 = W1.astype(f32)
    w2 = W2.astype(f32)
    gam = gamma.astype(f32).reshape(1, C)
    bet = beta.astype(f32).reshape(1, C)
    gmat = (jnp.arange(C)[:, None] // CG
            == jnp.arange(G)[None, :]).astype(f32)
    gmat_t = gmat.T
    offs = (jnp.arange(B, dtype=jnp.int32) * N)[:, None, None]
    idx_flat = (knn_inds.astype(jnp.int32) + offs).reshape(BN * K)
    filler = jnp.arange((BNP - BN) * K, dtype=jnp.int32) % BN
    idx_pad = jnp.concatenate([idx_flat, filler])

    edge, loc = _tc_feats(feature, w1, w2)
    s_aug = _sc_gather_sum(edge, idx_pad)
    a, c0 = _tc_stats(s_aug, loc, gam, bet, gmat, gmat_t)
    es, tt = _tc_tables(edge, loc, a, c0)
    out_t = _sc_edge_out(es, tt, idx_pad)
    return out_t[:BN].reshape(B, N, C).transpose(0, 2, 1)
